# Initial kernel scaffold; baseline (speedup 1.0000x reference)
import jax
import jax.numpy as jnp
from jax.experimental import pallas as pl


def kernel(queries, points, influ_scores, select_k):
    raise NotImplementedError("write your pallas kernel here")


# speedup vs baseline: 1.7216x; 1.7216x over previous
import functools

import jax
import jax.numpy as jnp
from jax import lax
from jax.experimental import pallas as pl
from jax.experimental.pallas import tpu as pltpu
from jax.experimental.pallas import tpu_sc as plsc

Q = 1024
K = 100000
SK = 16
L = 16
C = 6272
NCHUNK = 16
KP = C * NCHUNK
NW = 32
QPW = Q // NW
GRP = 8
MARGIN = 3e-5
NEG = -1e38

_SPLIT = jnp.float32(65537.0)
_DNUMS = lax.GatherDimensionNumbers(
    offset_dims=(), collapsed_slice_dims=(0,), start_index_map=(0,))

_SORT_STAGES = [(2, 1), (4, 2), (4, 1), (8, 4), (8, 2), (8, 1),
                (16, 8), (16, 4), (16, 2), (16, 1)]
_MERGE_STAGES = [(16, 8), (16, 4), (16, 2), (16, 1)]


def _bf16r(v):
    p = _SPLIT * v
    return p + (v - p)


def _sc_body(qflat_hbm, points_hbm, px_hbm, py_hbm, pz_hbm, inf_hbm,
             out_hbm, tv_hbm, ti_hbm,
             q_v, px_b, py_b, pz_b, inf_b, p2_b,
             topv, topi, tau, bqx, bqy, bqz, bq2,
             idxf, gx, gy, gz, outs, sem):
    it = lax.iota(jnp.int32, L)
    perm_idx = {d: (it ^ d)[:, None] for d in (1, 2, 4, 8)}
    def _ks(k, d):
        a = jnp.where((it & d) == 0, jnp.int32(1), jnp.int32(0))
        b = jnp.where((it & k) == 0, jnp.int32(1), jnp.int32(0))
        return jnp.where(a == b, jnp.int32(1), jnp.int32(0))

    ks_mask = {(k, d): _ks(k, d) for (k, d) in _SORT_STAGES}

    def _perm(v, idx):
        return lax.gather(v, idx, _DNUMS, (1,),
                          mode=lax.GatherScatterMode.PROMISE_IN_BOUNDS)

    def _allsum(v):
        for d in (1, 2, 4, 8):
            v = v + _perm(v, perm_idx[d])
        return v

    def _allmax(v):
        for d in (1, 2, 4, 8):
            v = jnp.maximum(v, _perm(v, perm_idx[d]))
        return v

    def _cmpex(kv, vv, stage):
        _, d = stage
        kp = _perm(kv, perm_idx[d])
        vp = _perm(vv, perm_idx[d])
        one = jnp.int32(1)
        zero = jnp.int32(0)
        lt = jnp.where(kv < kp, one, zero)
        eq = jnp.where(kv == kp, one, zero)
        gtv = jnp.where(vv > vp, one, zero)
        less32 = lt | (eq & gtv)
        take = less32 == ks_mask[stage]
        return jnp.where(take, kv, kp), jnp.where(take, vv, vp)

    def _bitonic_sort(kv, vv):
        for st in _SORT_STAGES:
            kv, vv = _cmpex(kv, vv, st)
        return kv, vv

    def _bitonic_merge(kv, vv):
        for st in _MERGE_STAGES:
            kv, vv = _cmpex(kv, vv, st)
        return kv, vv

    wid = lax.axis_index("s") * 2 + lax.axis_index("c")
    qbase = wid * QPW
    pltpu.sync_copy(qflat_hbm.at[pl.ds(qbase * 3, QPW * 3)],
                    q_v.at[pl.ds(0, QPW * 3)])

    neg = jnp.full((L,), NEG, jnp.float32)
    zero_i = jnp.zeros((L,), jnp.int32)

    def init_q(q, carry):
        sl = pl.ds(q * L, L)
        topv[sl] = neg
        topi[sl] = zero_i
        tau[sl] = neg
        qrow = q_v[pl.ds(q * 3, L)]
        qx = qrow[0]
        qy = qrow[1]
        qz = qrow[2]
        q2 = (qx * qx + qy * qy) + qz * qz
        bqx[sl] = _bf16r(jnp.broadcast_to(qx, (L,)))
        bqy[sl] = _bf16r(jnp.broadcast_to(qy, (L,)))
        bqz[sl] = _bf16r(jnp.broadcast_to(qz, (L,)))
        bq2[sl] = jnp.broadcast_to(q2, (L,))
        return carry

    lax.fori_loop(0, QPW, init_q, 0)

    def chunk_body(chunk, carry):
        cbase = chunk * C
        pltpu.sync_copy(px_hbm.at[pl.ds(cbase, C)], px_b)
        pltpu.sync_copy(py_hbm.at[pl.ds(cbase, C)], py_b)
        pltpu.sync_copy(pz_hbm.at[pl.ds(cbase, C)], pz_b)
        pltpu.sync_copy(inf_hbm.at[pl.ds(cbase, C)], inf_b)

        def prep(i, c2):
            sl = pl.ds(i * L, L)
            vx = px_b[sl]
            vy = py_b[sl]
            vz = pz_b[sl]
            p2_b[sl] = (vx * vx + vy * vy) + vz * vz
            px_b[sl] = _bf16r(vx)
            py_b[sl] = _bf16r(vy)
            pz_b[sl] = _bf16r(vz)
            return c2

        lax.fori_loop(0, C // L, prep, 0)

        def group_body(g, c3):
            bq = []
            for j in range(GRP):
                sl = pl.ds((g * GRP + j) * L, L)
                bq.append((bqx[sl], bqy[sl], bqz[sl], bq2[sl]))

            def exact_scores(bx, by, bz, q2v, vpx, vpy, vpz, vp2, vin):
                m0 = bx * vpx
                m1 = by * vpy
                m2 = bz * vpz
                s1 = m0 + m1
                ap = s1 - m1
                bp = s1 - ap
                e1 = (m0 - ap) + (m1 - bp)
                s2 = s1 + m2
                cp = s2 - m2
                dp = s2 - cp
                e2 = (s1 - cp) + (m2 - dp)
                m = s2 + (e1 + e2)
                d2 = (q2v + vp2) - (m + m)
                return vin - d2

            def step(t, c4):
                sl = pl.ds(t * L, L)
                vpx = px_b[sl]
                vpy = py_b[sl]
                vpz = pz_b[sl]
                vp2 = p2_b[sl]
                vin = inf_b[sl]
                exc = []
                for j in range(GRP):
                    bx, by, bz, q2v = bq[j]
                    m = (bx * vpx + by * vpy) + bz * vpz
                    d2 = (q2v + vp2) - (m + m)
                    s = vin - d2
                    exc.append(s - tau[pl.ds((g * GRP + j) * L, L)])
                emax = exc[0]
                for j in range(1, GRP):
                    emax = jnp.maximum(emax, exc[j])

                @pl.when(_allmax(emax)[0] > 0.0)
                def _():
                    gidx = it + (cbase + t * L)

                    def merge_q(j, c5):
                        q = g * GRP + j
                        sl2 = pl.ds(q * L, L)
                        bx = bqx[sl2]
                        by = bqy[sl2]
                        bz = bqz[sl2]
                        q2v = bq2[sl2]
                        se = exact_scores(bx, by, bz, q2v,
                                          vpx, vpy, vpz, vp2, vin)
                        ex = se - tau[sl2]

                        @pl.when(_allmax(ex)[0] > 0.0)
                        def _():
                            cv, ci = _bitonic_sort(se, gidx)
                            crv = lax.rev(cv, (0,))
                            cri = lax.rev(ci, (0,))
                            tvq = topv[sl2]
                            tiq = topi[sl2]
                            keep = tvq >= crv
                            lv = jnp.where(keep, tvq, crv)
                            li = jnp.where(keep, tiq, cri)
                            nv, ni = _bitonic_merge(lv, li)
                            topv[sl2] = nv
                            topi[sl2] = ni
                            tau[sl2] = jnp.broadcast_to(
                                nv[0] - MARGIN, (L,))

                        return c5

                    lax.fori_loop(0, GRP, merge_q, 0)

                return c4

            lax.fori_loop(0, C // L, step, 0)
            return c3

        lax.fori_loop(0, QPW // GRP, group_body, 0)
        return carry

    lax.fori_loop(0, NCHUNK, chunk_body, 0)

    def fin_q(q, carry):
        sl = pl.ds(q * L, L)
        topv[sl] = lax.rev(topv[sl], (0,))
        topi[sl] = lax.rev(topi[sl], (0,))
        idxf[sl] = topi[sl]
        return carry

    lax.fori_loop(0, QPW, fin_q, 0)

    pltpu.async_copy(px_hbm.at[idxf], gx, sem).wait()
    pltpu.async_copy(py_hbm.at[idxf], gy, sem).wait()
    pltpu.async_copy(pz_hbm.at[idxf], gz, sem).wait()

    zf = jnp.zeros((L,), jnp.float32)
    for blk in range(QPW // L):
        def out_q(j, accs, blk=blk):
            ax, ay, az = accs
            sl = pl.ds((blk * L + j) * L, L)
            tvq = topv[sl]
            e = jnp.exp(tvq - jnp.broadcast_to(tvq[0], (L,)))
            w = e / _allsum(e)
            ox = _allsum(w * gx[sl])
            oy = _allsum(w * gy[sl])
            oz = _allsum(w * gz[sl])
            lane = it == j
            return (jnp.where(lane, ox, ax),
                    jnp.where(lane, oy, ay),
                    jnp.where(lane, oz, az))

        ax, ay, az = lax.fori_loop(0, L, out_q, (zf, zf, zf))
        outs[pl.ds(blk * L, L)] = ax
        outs[pl.ds(QPW + blk * L, L)] = ay
        outs[pl.ds(2 * QPW + blk * L, L)] = az

    pltpu.sync_copy(topv.at[pl.ds(0, QPW * SK)],
                    tv_hbm.at[pl.ds(qbase * SK, QPW * SK)])
    pltpu.sync_copy(topi.at[pl.ds(0, QPW * SK)],
                    ti_hbm.at[pl.ds(qbase * SK, QPW * SK)])
    pltpu.sync_copy(outs.at[pl.ds(0, QPW)],
                    out_hbm.at[pl.ds(qbase, QPW)])
    pltpu.sync_copy(outs.at[pl.ds(QPW, QPW)],
                    out_hbm.at[pl.ds(Q + qbase, QPW)])
    pltpu.sync_copy(outs.at[pl.ds(2 * QPW, QPW)],
                    out_hbm.at[pl.ds(2 * Q + qbase, QPW)])


_papr_sc = functools.partial(
    pl.kernel,
    out_type=(
        jax.ShapeDtypeStruct((Q * 3,), jnp.float32),
        jax.ShapeDtypeStruct((Q * SK,), jnp.float32),
        jax.ShapeDtypeStruct((Q * SK,), jnp.int32),
    ),
    mesh=plsc.VectorSubcoreMesh(core_axis_name="c", subcore_axis_name="s"),
    scratch_types=[
        pltpu.VMEM((QPW * 3 + L,), jnp.float32),
        pltpu.VMEM((C,), jnp.float32),
        pltpu.VMEM((C,), jnp.float32),
        pltpu.VMEM((C,), jnp.float32),
        pltpu.VMEM((C,), jnp.float32),
        pltpu.VMEM((C,), jnp.float32),
        pltpu.VMEM((QPW * SK,), jnp.float32),
        pltpu.VMEM((QPW * SK,), jnp.int32),
        pltpu.VMEM((QPW * SK,), jnp.float32),
        pltpu.VMEM((QPW * SK,), jnp.float32),
        pltpu.VMEM((QPW * SK,), jnp.float32),
        pltpu.VMEM((QPW * SK,), jnp.float32),
        pltpu.VMEM((QPW * SK,), jnp.float32),
        pltpu.VMEM((QPW * SK,), jnp.int32),
        pltpu.VMEM((QPW * SK,), jnp.float32),
        pltpu.VMEM((QPW * SK,), jnp.float32),
        pltpu.VMEM((QPW * SK,), jnp.float32),
        pltpu.VMEM((QPW * 3,), jnp.float32),
        pltpu.SemaphoreType.DMA,
    ],
)(_sc_body)


def kernel(queries, points, influ_scores, select_k):
    pad = KP - K
    px = jnp.pad(points[:, 0], (0, pad))
    py = jnp.pad(points[:, 1], (0, pad))
    pz = jnp.pad(points[:, 2], (0, pad))
    inf = jnp.pad(influ_scores[:, 0], (0, pad), constant_values=-1e30)
    out_flat, tv, ti = _papr_sc(queries.reshape(-1), points, px, py, pz, inf)
    return (out_flat.reshape(3, Q).T, tv.reshape(Q, SK), ti.reshape(Q, SK))
